# Initial kernel scaffold; baseline (speedup 1.0000x reference)
import jax
import jax.numpy as jnp
from jax.experimental import pallas as pl


def kernel(values, edge_index, weights):
    raise NotImplementedError("write your pallas kernel here")


# speedup vs baseline: 246.4731x; 246.4731x over previous
import functools

import jax
import jax.numpy as jnp
from jax import lax
from jax.experimental import pallas as pl
from jax.experimental.pallas import tpu as pltpu
from jax.experimental.pallas import tpu_sc as plsc

N = 100000
E = 6400000
T = 3
LR = 0.1
N_SENSORY = 784

NC, NS, L = 2, 16, 16
TILES = NC * NS
EPT = E // TILES
C = 4000
NCHUNK = EPT // C
ROWS, COLS = 784, 128
NPAD = ROWS * COLS
SLICE = NPAD // NS


def _seg_body(g_hbm, gidx_hbm, sidx_hbm, w_hbm, out_hbm,
              g_loc, gi_v, si_v, w_v, contrib_v, acc):
    c = lax.axis_index("c")
    s = lax.axis_index("s")
    wid = c * NS + s

    def _zero(i, _):
        contrib_v[pl.ds(i * 16, 16)] = jnp.zeros((16,), jnp.float32)
        return 0
    lax.fori_loop(0, C // 16, _zero, 0)
    base_a = pl.multiple_of(s * SLICE, 8)
    pltpu.sync_copy(contrib_v.at[pl.ds(0, C)], acc.at[pl.ds(base_a, C)])
    pltpu.sync_copy(contrib_v.at[pl.ds(0, SLICE - C)],
                    acc.at[pl.ds(base_a + C, SLICE - C)])

    pltpu.sync_copy(g_hbm.at[pl.ds(0, N)], g_loc)
    plsc.subcore_barrier()

    def _chunk(gi, _):
        base = pl.multiple_of(wid * EPT + gi * C, 8)
        pltpu.sync_copy(gidx_hbm.at[pl.ds(base, C)], gi_v)
        pltpu.sync_copy(sidx_hbm.at[pl.ds(base, C)], si_v)
        pltpu.sync_copy(w_hbm.at[pl.ds(base, C)], w_v)

        def _inner(i, _):
            ii = pl.multiple_of(i * 16, 16)
            idx16 = gi_v[pl.ds(ii, 16)]
            vals = plsc.load_gather(g_loc, [idx16])
            contrib_v[pl.ds(ii, 16)] = w_v[pl.ds(ii, 16)] * vals
            return 0
        lax.fori_loop(0, C // 16, _inner, 0)
        pltpu.sync_copy(contrib_v, acc.at[si_v], add=True)
        return 0
    lax.fori_loop(0, NCHUNK, _chunk, 0)

    plsc.subcore_barrier()
    pltpu.sync_copy(acc.at[pl.ds(base_a, SLICE)],
                    out_hbm.at[c, pl.ds(base_a, SLICE)])


_seg_sum = functools.partial(
    pl.kernel,
    out_type=jax.ShapeDtypeStruct((NC, NPAD), jnp.float32),
    mesh=plsc.VectorSubcoreMesh(core_axis_name="c", subcore_axis_name="s"),
    compiler_params=pltpu.CompilerParams(needs_layout_passes=False),
    scratch_types=[
        pltpu.VMEM((N,), jnp.float32),
        pltpu.VMEM((C,), jnp.int32),
        pltpu.VMEM((C,), jnp.int32),
        pltpu.VMEM((C,), jnp.float32),
        pltpu.VMEM((C,), jnp.float32),
        pltpu.VMEM_SHARED((NPAD,), jnp.float32),
    ],
)(_seg_body)


def _tanh_body(x_ref, o_ref):
    o_ref[...] = jnp.tanh(x_ref[...])


_tanh_tc = pl.pallas_call(
    _tanh_body, out_shape=jax.ShapeDtypeStruct((ROWS, COLS), jnp.float32))


def _err_body(v_ref, p_ref, o_ref):
    o_ref[...] = v_ref[...] - p_ref[0] - p_ref[1]


_err_tc = pl.pallas_call(
    _err_body, out_shape=jax.ShapeDtypeStruct((ROWS, COLS), jnp.float32))


def _upd_body(v_ref, fv_ref, e_ref, b_ref, vn_ref, fvn_ref):
    idx = (lax.broadcasted_iota(jnp.int32, (ROWS, COLS), 0) * COLS
           + lax.broadcasted_iota(jnp.int32, (ROWS, COLS), 1))
    mask = (idx >= N_SENSORY).astype(jnp.float32)
    fv = fv_ref[...]
    back = b_ref[0] + b_ref[1]
    grad = e_ref[...] - (1.0 - fv * fv) * back
    vn = v_ref[...] - LR * mask * grad
    vn_ref[...] = vn
    fvn_ref[...] = jnp.tanh(vn)


_upd_tc = pl.pallas_call(
    _upd_body,
    out_shape=(jax.ShapeDtypeStruct((ROWS, COLS), jnp.float32),
               jax.ShapeDtypeStruct((ROWS, COLS), jnp.float32)))


def kernel(values, edge_index, weights):
    src = edge_index[0]
    dst = edge_index[1]
    v2 = jnp.pad(values, (0, NPAD - N)).reshape(ROWS, COLS)
    fv2 = _tanh_tc(v2)
    for _ in range(T):
        preds = _seg_sum(fv2.reshape(NPAD), src, dst, weights)
        err2 = _err_tc(v2, preds.reshape(NC, ROWS, COLS))
        back = _seg_sum(err2.reshape(NPAD), dst, src, weights)
        v2, fv2 = _upd_tc(v2, fv2, err2, back.reshape(NC, ROWS, COLS))
    return v2.reshape(-1)[:N]


# speedup vs baseline: 420.5657x; 1.7063x over previous
import functools

import jax
import jax.numpy as jnp
from jax import lax
from jax.experimental import pallas as pl
from jax.experimental.pallas import tpu as pltpu
from jax.experimental.pallas import tpu_sc as plsc

N = 100000
E = 6400000
T = 3
LR = 0.1
N_SENSORY = 784

NC, NS, L = 2, 16, 16
TILES = NC * NS
EPT = E // TILES
C = 4000
NCHUNK = EPT // C
ROWS, COLS = 784, 128
NPAD = ROWS * COLS
SLICE = NPAD // NS


def _seg_body(g_hbm, gidx_hbm, sidx_hbm, w_hbm, out_hbm,
              g_loc, gi_v, si0, si1, w_v, co0, co1, acc, in_sem, sc_sem):
    c = lax.axis_index("c")
    s = lax.axis_index("s")
    wid = c * NS + s

    def _zero(i, _):
        co0[pl.ds(i * 16, 16)] = jnp.zeros((16,), jnp.float32)
        return 0
    lax.fori_loop(0, C // 16, _zero, 0)
    base_a = pl.multiple_of(s * SLICE, 8)
    pltpu.sync_copy(co0.at[pl.ds(0, C)], acc.at[pl.ds(base_a, C)])
    pltpu.sync_copy(co0.at[pl.ds(0, SLICE - C)],
                    acc.at[pl.ds(base_a + C, SLICE - C)])

    pltpu.sync_copy(g_hbm.at[pl.ds(0, N)], g_loc)
    plsc.subcore_barrier()

    def _prep_fire(gi, si_v, co_v):
        base = pl.multiple_of(wid * EPT + gi * C, 8)
        d1 = pltpu.async_copy(gidx_hbm.at[pl.ds(base, C)], gi_v, in_sem)
        d2 = pltpu.async_copy(sidx_hbm.at[pl.ds(base, C)], si_v, in_sem)
        d3 = pltpu.async_copy(w_hbm.at[pl.ds(base, C)], w_v, in_sem)
        d1.wait()
        d2.wait()
        d3.wait()

        def _inner(i, _):
            ii = pl.multiple_of(i * 16, 16)
            idx16 = gi_v[pl.ds(ii, 16)]
            vals = plsc.load_gather(g_loc, [idx16])
            co_v[pl.ds(ii, 16)] = w_v[pl.ds(ii, 16)] * vals
            return 0
        lax.fori_loop(0, C // 16, _inner, 0)
        pltpu.async_copy(co_v, acc.at[si_v], sc_sem, add=True)

    def _wait_scatter(si_v, co_v):
        pltpu.make_async_copy(co_v, acc.at[si_v], sc_sem).wait()

    _prep_fire(0, si0, co0)
    _prep_fire(1, si1, co1)

    def _pair(gg, _):
        _wait_scatter(si0, co0)
        _prep_fire(2 * gg, si0, co0)
        _wait_scatter(si1, co1)
        _prep_fire(2 * gg + 1, si1, co1)
        return 0
    lax.fori_loop(1, NCHUNK // 2, _pair, 0)
    _wait_scatter(si0, co0)
    _wait_scatter(si1, co1)

    plsc.subcore_barrier()
    pltpu.sync_copy(acc.at[pl.ds(base_a, SLICE)],
                    out_hbm.at[c, pl.ds(base_a, SLICE)])


_seg_sum = functools.partial(
    pl.kernel,
    out_type=jax.ShapeDtypeStruct((NC, NPAD), jnp.float32),
    mesh=plsc.VectorSubcoreMesh(core_axis_name="c", subcore_axis_name="s"),
    compiler_params=pltpu.CompilerParams(needs_layout_passes=False),
    scratch_types=[
        pltpu.VMEM((N,), jnp.float32),
        pltpu.VMEM((C,), jnp.int32),
        pltpu.VMEM((C,), jnp.int32),
        pltpu.VMEM((C,), jnp.int32),
        pltpu.VMEM((C,), jnp.float32),
        pltpu.VMEM((C,), jnp.float32),
        pltpu.VMEM((C,), jnp.float32),
        pltpu.VMEM_SHARED((NPAD,), jnp.float32),
        pltpu.SemaphoreType.DMA,
        pltpu.SemaphoreType.DMA,
    ],
)(_seg_body)


def _tanh_body(x_ref, o_ref):
    o_ref[...] = jnp.tanh(x_ref[...])


_tanh_tc = pl.pallas_call(
    _tanh_body, out_shape=jax.ShapeDtypeStruct((ROWS, COLS), jnp.float32))


def _err_body(v_ref, p_ref, o_ref):
    o_ref[...] = v_ref[...] - p_ref[0] - p_ref[1]


_err_tc = pl.pallas_call(
    _err_body, out_shape=jax.ShapeDtypeStruct((ROWS, COLS), jnp.float32))


def _upd_body(v_ref, fv_ref, e_ref, b_ref, vn_ref, fvn_ref):
    idx = (lax.broadcasted_iota(jnp.int32, (ROWS, COLS), 0) * COLS
           + lax.broadcasted_iota(jnp.int32, (ROWS, COLS), 1))
    mask = (idx >= N_SENSORY).astype(jnp.float32)
    fv = fv_ref[...]
    back = b_ref[0] + b_ref[1]
    grad = e_ref[...] - (1.0 - fv * fv) * back
    vn = v_ref[...] - LR * mask * grad
    vn_ref[...] = vn
    fvn_ref[...] = jnp.tanh(vn)


_upd_tc = pl.pallas_call(
    _upd_body,
    out_shape=(jax.ShapeDtypeStruct((ROWS, COLS), jnp.float32),
               jax.ShapeDtypeStruct((ROWS, COLS), jnp.float32)))


def kernel(values, edge_index, weights):
    src = edge_index[0]
    dst = edge_index[1]
    v2 = jnp.pad(values, (0, NPAD - N)).reshape(ROWS, COLS)
    fv2 = _tanh_tc(v2)
    for _ in range(T):
        preds = _seg_sum(fv2.reshape(NPAD), src, dst, weights)
        err2 = _err_tc(v2, preds.reshape(NC, ROWS, COLS))
        back = _seg_sum(err2.reshape(NPAD), dst, src, weights)
        v2, fv2 = _upd_tc(v2, fv2, err2, back.reshape(NC, ROWS, COLS))
    return v2.reshape(-1)[:N]
